# Initial kernel scaffold; baseline (speedup 1.0000x reference)
import jax
import jax.numpy as jnp
from jax.experimental import pallas as pl


def kernel(input, edge_index, W0, b0, W1, b1):
    raise NotImplementedError("write your pallas kernel here")


# speedup vs baseline: 10.1098x; 10.1098x over previous
import functools

import jax
import jax.numpy as jnp
from jax import lax
from jax.experimental import pallas as pl
from jax.experimental.pallas import tpu as pltpu
from jax.experimental.pallas import tpu_sc as plsc

F = 16
ROW = 128
CH = 16
NC = 2
NS = 16
NW = NC * NS


@functools.lru_cache(maxsize=None)
def _make_seg_kernel(n: int, chunks: int, acc_rows: int, erows: int):
    half = n // NC
    zrows = acc_rows // NS
    orows = -(-(half // NS) // 8) * 8
    olast = half - (NS - 1) * orows
    assert olast > 0 and olast % 8 == 0 and zrows % 8 == 0
    mesh = plsc.VectorSubcoreMesh(core_axis_name="c", subcore_axis_name="s")

    def body(x_hbm, srcr_hbm, dstr_hbm, init_hbm, out_hbm,
             src_v, dst_v, rows_v, acc_sh, gsem):
        c = lax.axis_index("c")
        s = lax.axis_index("s")
        pltpu.sync_copy(init_hbm,
                        acc_sh.at[pl.ds(pl.multiple_of(s * zrows, 8), zrows)])
        plsc.subcore_barrier()

        def chunk(i, carry):
            base = (s * chunks + i) * CH
            pltpu.sync_copy(srcr_hbm.at[pl.ds(base, CH)], src_v)
            pltpu.sync_copy(dstr_hbm.at[pl.ds(c * erows + base, CH)], dst_v)
            cps = [pltpu.async_copy(x_hbm.at[src_v.at[j]], rows_v.at[j], gsem)
                   for j in range(CH)]
            for cp in cps:
                cp.wait()
            for j in range(CH):
                pltpu.sync_copy(rows_v.at[j], acc_sh.at[dst_v.at[j]], add=True)
            return carry

        lax.fori_loop(0, chunks, chunk, 0)
        plsc.subcore_barrier()
        obase = pl.multiple_of(c * half + s * orows, 8)

        @pl.when(s < NS - 1)
        def _copy_full():
            pltpu.sync_copy(acc_sh.at[pl.ds(pl.multiple_of(s * orows, 8), orows)],
                            out_hbm.at[pl.ds(obase, orows)])

        @pl.when(s == NS - 1)
        def _copy_last():
            pltpu.sync_copy(acc_sh.at[pl.ds((NS - 1) * orows, olast)],
                            out_hbm.at[pl.ds(obase, olast)])

    return pl.kernel(
        body,
        out_type=jax.ShapeDtypeStruct((n, F), jnp.float32),
        mesh=mesh,
        scratch_types=[
            pltpu.VMEM((CH, ROW), jnp.int32),
            pltpu.VMEM((CH, ROW), jnp.int32),
            pltpu.VMEM((CH, ROW, F), jnp.float32),
            pltpu.VMEM_SHARED((acc_rows, F), jnp.float32),
            pltpu.SemaphoreType.DMA,
        ],
        compiler_params=pltpu.CompilerParams(use_tc_tiling_on_sc=False),
    )


def _fused_mlp(agg, W0, b0, W1, n, br=1000):
    mid = W0.shape[1]
    grid = n // br

    def mm_body(p_ref, w0_ref, b0_ref, w1_ref, o_ref):
        h = jnp.dot(p_ref[...], w0_ref[...], preferred_element_type=jnp.float32)
        h = jnp.maximum(h + b0_ref[...], 0.0)
        o_ref[...] = jnp.dot(h, w1_ref[...], preferred_element_type=jnp.float32)

    return pl.pallas_call(
        mm_body,
        grid=(grid,),
        in_specs=[
            pl.BlockSpec((br, F), lambda i: (i, 0)),
            pl.BlockSpec((F, mid), lambda i: (0, 0)),
            pl.BlockSpec((1, mid), lambda i: (0, 0)),
            pl.BlockSpec((mid, F), lambda i: (0, 0)),
        ],
        out_specs=pl.BlockSpec((br, F), lambda i: (i, 0)),
        out_shape=jax.ShapeDtypeStruct((n, F), jnp.float32),
    )(agg, W0, b0.reshape(1, mid), W1)


def kernel(input, edge_index, W0, b0, W1, b1):
    n, f = input.shape
    assert f == F and n % (NC * 8) == 0
    e = edge_index.shape[1]
    half = n // NC
    per = NS * CH * ROW
    chunks = -(-e // per)
    e_pad = chunks * per
    pad = e_pad - e
    erows = e_pad // ROW
    src = jnp.concatenate([edge_index[0], jnp.zeros((pad,), jnp.int32)])
    dst = jnp.concatenate([edge_index[1], jnp.full((pad,), n, jnp.int32)])
    dst0 = jnp.where(dst < half, dst, half)
    dst1 = jnp.where(dst >= half, dst - half, half)
    dst1 = jnp.minimum(dst1, half)
    srcr = src.reshape(erows, ROW)
    dstr = jnp.concatenate([dst0, dst1]).reshape(2 * erows, ROW)
    acc_rows = -(-(half + 1) // (NS * 8)) * NS * 8
    zrows = acc_rows // NS

    seg = _make_seg_kernel(n, chunks, acc_rows, erows)
    zeros = jnp.zeros((zrows, F), jnp.float32)
    agg0 = seg(input, srcr, dstr, zeros)
    h2 = _fused_mlp(agg0, W0, b0, W1, n)
    binit = jnp.broadcast_to(b1.reshape(1, F), (zrows, F))
    return seg(h2, srcr, dstr, binit)


# speedup vs baseline: 10.1318x; 1.0022x over previous
import functools

import jax
import jax.numpy as jnp
from jax import lax
from jax.experimental import pallas as pl
from jax.experimental.pallas import tpu as pltpu
from jax.experimental.pallas import tpu_sc as plsc

F = 16
EC = 2048
NC = 2
NS = 16
NW = NC * NS


@functools.lru_cache(maxsize=None)
def _make_seg_kernel(n: int, chunks: int, acc_rows: int, epad: int):
    half = n // NC
    zrows = acc_rows // NS
    orows = -(-(half // NS) // 8) * 8
    olast = half - (NS - 1) * orows
    assert olast > 0 and olast % 8 == 0 and zrows % 8 == 0
    mesh = plsc.VectorSubcoreMesh(core_axis_name="c", subcore_axis_name="s")

    def body(x_hbm, src_hbm, dst_hbm, init_hbm, out_hbm,
             src_v, dst_v, rows_v, acc_sh, gsem):
        c = lax.axis_index("c")
        s = lax.axis_index("s")
        pltpu.sync_copy(init_hbm,
                        acc_sh.at[pl.ds(pl.multiple_of(s * zrows, 8), zrows)])
        plsc.subcore_barrier()

        def chunk(i, carry):
            base = (s * chunks + i) * EC
            pltpu.sync_copy(src_hbm.at[pl.ds(base, EC)], src_v)
            pltpu.sync_copy(dst_hbm.at[pl.ds(c * epad + base, EC)], dst_v)
            pltpu.async_copy(x_hbm.at[src_v], rows_v, gsem).wait()
            pltpu.sync_copy(rows_v, acc_sh.at[dst_v], add=True)
            return carry

        lax.fori_loop(0, chunks, chunk, 0)
        plsc.subcore_barrier()
        obase = pl.multiple_of(c * half + s * orows, 8)

        @pl.when(s < NS - 1)
        def _copy_full():
            pltpu.sync_copy(acc_sh.at[pl.ds(pl.multiple_of(s * orows, 8), orows)],
                            out_hbm.at[pl.ds(obase, orows)])

        @pl.when(s == NS - 1)
        def _copy_last():
            pltpu.sync_copy(acc_sh.at[pl.ds((NS - 1) * orows, olast)],
                            out_hbm.at[pl.ds(obase, olast)])

    return pl.kernel(
        body,
        out_type=jax.ShapeDtypeStruct((n, F), jnp.float32),
        mesh=mesh,
        scratch_types=[
            pltpu.VMEM((EC,), jnp.int32),
            pltpu.VMEM((EC,), jnp.int32),
            pltpu.VMEM((EC, F), jnp.float32),
            pltpu.VMEM_SHARED((acc_rows, F), jnp.float32),
            pltpu.SemaphoreType.DMA,
        ],
        compiler_params=pltpu.CompilerParams(use_tc_tiling_on_sc=False),
    )


def _fused_mlp(agg, W0, b0, W1, n, br=1000):
    mid = W0.shape[1]
    grid = n // br

    def mm_body(p_ref, w0_ref, b0_ref, w1_ref, o_ref):
        h = jnp.dot(p_ref[...], w0_ref[...], preferred_element_type=jnp.float32)
        h = jnp.maximum(h + b0_ref[...], 0.0)
        o_ref[...] = jnp.dot(h, w1_ref[...], preferred_element_type=jnp.float32)

    return pl.pallas_call(
        mm_body,
        grid=(grid,),
        in_specs=[
            pl.BlockSpec((br, F), lambda i: (i, 0)),
            pl.BlockSpec((F, mid), lambda i: (0, 0)),
            pl.BlockSpec((1, mid), lambda i: (0, 0)),
            pl.BlockSpec((mid, F), lambda i: (0, 0)),
        ],
        out_specs=pl.BlockSpec((br, F), lambda i: (i, 0)),
        out_shape=jax.ShapeDtypeStruct((n, F), jnp.float32),
    )(agg, W0, b0.reshape(1, mid), W1)


def kernel(input, edge_index, W0, b0, W1, b1):
    n, f = input.shape
    assert f == F and n % (NC * 8) == 0
    e = edge_index.shape[1]
    half = n // NC
    per = NS * EC
    chunks = -(-e // per)
    e_pad = chunks * per
    pad = e_pad - e
    src = jnp.concatenate([edge_index[0], jnp.zeros((pad,), jnp.int32)])
    dst = jnp.concatenate([edge_index[1], jnp.full((pad,), n, jnp.int32)])
    dst0 = jnp.where(dst < half, dst, half)
    dst1 = jnp.where(dst >= half, dst - half, half)
    dst1 = jnp.minimum(dst1, half)
    srcr = src
    dstr = jnp.concatenate([dst0, dst1])
    acc_rows = -(-(half + 1) // (NS * 8)) * NS * 8
    zrows = acc_rows // NS

    seg = _make_seg_kernel(n, chunks, acc_rows, e_pad)
    zeros = jnp.zeros((zrows, F), jnp.float32)
    agg0 = seg(input, srcr, dstr, zeros)
    h2 = _fused_mlp(agg0, W0, b0, W1, n)
    binit = jnp.broadcast_to(b1.reshape(1, F), (zrows, F))
    return seg(h2, srcr, dstr, binit)
